# baseline (speedup 1.0000x reference)
import functools

import jax
import jax.numpy as jnp
from jax import lax
from jax.experimental import pallas as pl
from jax.experimental.pallas import tpu as pltpu
from jax.experimental.pallas import tpu_sc as plsc

_VOCAB = 1000000
_EMB = 64
_CTX = 16
_RF = 8
_B = 16384
_T = _CTX + _RF - 1


def _make_sc_gather(Vp, Dp, B):
    info = plsc.get_sparse_core_info()
    NC, NS = info.num_cores, info.num_subcores
    NW = NC * NS
    b_per_w = B // NW
    mesh = plsc.VectorSubcoreMesh(core_axis_name="c", subcore_axis_name="s")

    @functools.partial(
        pl.kernel,
        mesh=mesh,
        out_type=jax.ShapeDtypeStruct((B, Dp), jnp.float32),
        scratch_types=[
            pltpu.VMEM((b_per_w,), jnp.int32),
            pltpu.VMEM((b_per_w, Dp), jnp.float32),
            pltpu.SemaphoreType.DMA,
        ],
    )
    def gather_k(table_hbm, idx_hbm, out_hbm, idx_v, rows_v, sem):
        wid = lax.axis_index("s") * NC + lax.axis_index("c")
        base = wid * b_per_w
        pltpu.sync_copy(idx_hbm.at[pl.ds(base, b_per_w)], idx_v)
        for g in range(b_per_w // 16):
            idx_v[pl.ds(g * 16, 16)] = (
                idx_v[pl.ds(g * 16, 16)] >> 1
            )
        pltpu.async_copy(table_hbm.at[idx_v], rows_v, sem).wait()
        pltpu.sync_copy(rows_v, out_hbm.at[pl.ds(base, b_per_w)])

    return gather_k


def _tc_body(state_ref, emb2_ref, idx_ref, wc_ref, out_ref):
    s = state_ref[...]
    st = jnp.swapaxes(s, 1, 2)
    par = (idx_ref[...] & 1).astype(jnp.bool_)
    e2 = emb2_ref[...]
    e = jnp.where(par, e2[:, _EMB:], e2[:, :_EMB])
    inv_rf = 1.0 / _RF
    ys = []
    for j in range(_CTX):
        x = e if j == 0 else st[:, j - 1, :]
        m = jnp.sum(st[:, j : j + _RF, :], axis=1) * inv_rf
        c = jnp.concatenate([x, m], axis=1)
        y = jnp.tanh(jnp.dot(c, wc_ref[j], preferred_element_type=jnp.float32))
        ys.append(y)
    Y = jnp.stack(ys, axis=1)
    outt = jnp.concatenate([Y, st[:, _CTX:, :]], axis=1)
    out_ref[...] = jnp.swapaxes(outt, 1, 2)


def _tc_call(network_state, emb2, input_indices, Wc, bs):
    n_blocks = _B // bs
    return pl.pallas_call(
        _tc_body,
        grid=(n_blocks,),
        in_specs=[
            pl.BlockSpec((bs, _EMB, _T), lambda i: (i, 0, 0)),
            pl.BlockSpec((bs, 2 * _EMB), lambda i: (i, 0)),
            pl.BlockSpec((bs, 1), lambda i: (i, 0)),
            pl.BlockSpec((_CTX, 2 * _EMB, _EMB), lambda i: (0, 0, 0)),
        ],
        out_specs=pl.BlockSpec((bs, _EMB, _T), lambda i: (i, 0, 0)),
        out_shape=jax.ShapeDtypeStruct((_B, _EMB, _T), jnp.float32),
        compiler_params=pltpu.CompilerParams(
            dimension_semantics=("arbitrary",),
        ),
    )(network_state, emb2, input_indices, Wc)


def kernel(emb_table, network_state, W1, W2, input_indices):
    idx = input_indices[:, 0]
    table_packed = emb_table.reshape(_VOCAB // 2, 2 * _EMB)
    emb2 = _make_sc_gather(_VOCAB // 2, 2 * _EMB, _B)(table_packed, idx)
    Wc = jnp.concatenate([W1, W2], axis=1)
    return _tc_call(network_state, emb2, input_indices, Wc, bs=128)


# speedup vs baseline: 2.7393x; 2.7393x over previous
import functools

import jax
import jax.numpy as jnp
from jax import lax
from jax.experimental import pallas as pl
from jax.experimental.pallas import tpu as pltpu
from jax.experimental.pallas import tpu_sc as plsc

_VOCAB = 1000000
_EMB = 64
_CTX = 16
_RF = 8
_B = 16384
_T = _CTX + _RF - 1
_IC = 128


def _make_sc_gather(B):
    info = plsc.get_sparse_core_info()
    NC, NS = info.num_cores, info.num_subcores
    NW = NC * NS
    b_per_w = B // NW
    mesh = plsc.VectorSubcoreMesh(core_axis_name="c", subcore_axis_name="s")

    @functools.partial(
        pl.kernel,
        mesh=mesh,
        out_type=jax.ShapeDtypeStruct((B, _EMB), jnp.float32),
        scratch_types=[
            pltpu.VMEM((b_per_w,), jnp.int32),
            pltpu.VMEM((b_per_w, _EMB), jnp.float32),
            pltpu.SemaphoreType.DMA,
        ],
        compiler_params=pltpu.CompilerParams(use_tc_tiling_on_sc=False),
    )
    def gather_k(table_hbm, idx_hbm, out_hbm, idx_v, rows_v, sem):
        wid = lax.axis_index("s") * NC + lax.axis_index("c")
        base = wid * b_per_w
        pltpu.sync_copy(idx_hbm.at[pl.ds(base, b_per_w)], idx_v)
        copies = []
        for q in range(b_per_w // _IC):
            sl = pl.ds(q * _IC, _IC)
            copies.append(
                pltpu.async_copy(table_hbm.at[idx_v.at[sl]], rows_v.at[sl], sem)
            )
        for cp in copies:
            cp.wait()
        pltpu.sync_copy(rows_v, out_hbm.at[pl.ds(base, b_per_w)])

    return gather_k


def _tc_body(st_ref, emb_ref, wct_ref, out_ref):
    st = st_ref[...]
    inv_rf = 1.0 / _RF
    w = st[0]
    for t in range(1, _RF):
        w = w + st[t]
    for j in range(_CTX):
        m = w * inv_rf
        if j == 0:
            y = jnp.tanh(
                lax.dot_general(
                    wct_ref[0, :, :_EMB],
                    emb_ref[...],
                    (((1,), (1,)), ((), ())),
                    preferred_element_type=jnp.float32,
                )
                + jnp.dot(
                    wct_ref[0, :, _EMB:], m, preferred_element_type=jnp.float32
                )
            )
        else:
            c = jnp.concatenate([st[j - 1], m], axis=0)
            y = jnp.tanh(
                jnp.dot(wct_ref[j], c, preferred_element_type=jnp.float32)
            )
        out_ref[j] = y
        if j + _RF < _T:
            w = w - st[j] + st[j + _RF]
    out_ref[_CTX:] = st[_CTX:]


def _tc_call(st_t, emb, WcT, bn):
    n_blocks = _B // bn
    return pl.pallas_call(
        _tc_body,
        grid=(n_blocks,),
        in_specs=[
            pl.BlockSpec((_T, _EMB, bn), lambda i: (0, 0, i)),
            pl.BlockSpec((bn, _EMB), lambda i: (i, 0)),
            pl.BlockSpec((_CTX, _EMB, 2 * _EMB), lambda i: (0, 0, 0)),
        ],
        out_specs=pl.BlockSpec((_T, _EMB, bn), lambda i: (0, 0, i)),
        out_shape=jax.ShapeDtypeStruct((_T, _EMB, _B), jnp.float32),
        compiler_params=pltpu.CompilerParams(
            dimension_semantics=("arbitrary",),
        ),
    )(st_t, emb, WcT)


def kernel(emb_table, network_state, W1, W2, input_indices):
    idx = input_indices[:, 0]
    emb = _make_sc_gather(_B)(emb_table, idx)
    WcT = jnp.transpose(jnp.concatenate([W1, W2], axis=1), (0, 2, 1))
    st_t = jnp.transpose(network_state, (2, 1, 0))
    out_t = _tc_call(st_t, emb, WcT, bn=512)
    return jnp.transpose(out_t, (2, 1, 0))
